# Initial kernel scaffold; baseline (speedup 1.0000x reference)
import jax
import jax.numpy as jnp
from jax.experimental import pallas as pl


def kernel(x, pos, batch, params):
    raise NotImplementedError("write your pallas kernel here")


# speedup vs baseline: 8.9802x; 8.9802x over previous
import functools

import numpy as np
import jax
import jax.numpy as jnp
from jax import lax
from jax.experimental import pallas as pl
from jax.experimental.pallas import tpu as pltpu
from jax.experimental.pallas import tpu_sc as plsc

MAXNB = 64
_NEG_INF = float("-inf")

_pallas_call = pl.pallas_call


def _fps_kernel(px_ref, py_ref, pz_ref, idx_ref, sx_ref, sy_ref, sz_ref, *,
                n, N, R):
    fi = (lax.broadcasted_iota(jnp.int32, (R, 128), 0) * 128
          + lax.broadcasted_iota(jnp.int32, (R, 128), 1))
    lane = lax.broadcasted_iota(jnp.int32, (1, 128), 1)

    def extract(ref, row, col):
        rv = ref[pl.ds(row, 1), :]
        return jnp.sum(jnp.where(lane == col, rv, 0.0))

    px = px_ref[...]
    py = py_ref[...]
    pz = pz_ref[...]

    lx0 = extract(px_ref, 0, 0)
    ly0 = extract(py_ref, 0, 0)
    lz0 = extract(pz_ref, 0, 0)
    idx_ref[pl.ds(0, 1), :] = jnp.zeros((1, 1), jnp.int32)
    sx_ref[pl.ds(0, 1), :] = jnp.reshape(lx0, (1, 1))
    sy_ref[pl.ds(0, 1), :] = jnp.reshape(ly0, (1, 1))
    sz_ref[pl.ds(0, 1), :] = jnp.reshape(lz0, (1, 1))

    def body(i, carry):
        dmin, lx, ly, lz = carry
        dx = px - lx
        dy = py - ly
        dz = pz - lz
        d = dx * dx + dy * dy + dz * dz
        dmin = jnp.minimum(dmin, d)
        m = jnp.max(dmin)
        bid = jnp.min(jnp.where(dmin == m, fi, N))
        idx_ref[pl.ds(i, 1), :] = jnp.reshape(bid, (1, 1))
        row = bid // 128
        col = bid % 128
        nlx = extract(px_ref, row, col)
        nly = extract(py_ref, row, col)
        nlz = extract(pz_ref, row, col)
        sx_ref[pl.ds(i, 1), :] = jnp.reshape(nlx, (1, 1))
        sy_ref[pl.ds(i, 1), :] = jnp.reshape(nly, (1, 1))
        sz_ref[pl.ds(i, 1), :] = jnp.reshape(nlz, (1, 1))
        return dmin, nlx, nly, nlz

    dmin0 = jnp.full((R, 128), jnp.inf, jnp.float32)
    lax.fori_loop(1, n, body, (dmin0, lx0, ly0, lz0))


def _fps(pos, n, interpret=False):
    N = pos.shape[0]
    R = N // 128
    px = pos[:, 0].reshape(R, 128)
    py = pos[:, 1].reshape(R, 128)
    pz = pos[:, 2].reshape(R, 128)
    outs = _pallas_call(
        functools.partial(_fps_kernel, n=n, N=N, R=R),
        out_shape=(jax.ShapeDtypeStruct((n, 1), jnp.int32),
                   jax.ShapeDtypeStruct((n, 1), jnp.float32),
                   jax.ShapeDtypeStruct((n, 1), jnp.float32),
                   jax.ShapeDtypeStruct((n, 1), jnp.float32)),
        interpret=interpret,
    )(px, py, pz)
    return outs


def _radius_kernel(posd_ref, src3_ref, nb_ref, valid_ref, *, N, NC, r2):
    QB = posd_ref.shape[0]
    pd = posd_ref[...]
    qn = jnp.sum(pd * pd, axis=1, keepdims=True)
    nch = N // NC
    U = (lax.broadcasted_iota(jnp.int32, (NC, NC), 0)
         <= lax.broadcasted_iota(jnp.int32, (NC, NC), 1)).astype(jnp.float32)
    thr = lax.broadcasted_iota(jnp.int32, (1, MAXNB), 1)

    def chunk(c, carry):
        acc, prior = carry
        st = src3_ref[pl.ds(c, 1), :, :].reshape(3, NC)
        pn = jnp.sum(st * st, axis=0, keepdims=True)
        d2 = qn + pn - 2.0 * jnp.dot(pd, st, preferred_element_type=jnp.float32)
        win = (d2 <= r2).astype(jnp.float32)
        cum = prior + jnp.dot(win, U, precision=lax.Precision.HIGHEST,
                              preferred_element_type=jnp.float32)
        cols = [jnp.sum((cum <= j).astype(jnp.float32), axis=1, keepdims=True)
                for j in range(MAXNB)]
        acc = acc + jnp.concatenate(cols, axis=1)
        prior = cum[:, NC - 1:NC]
        return acc, prior

    acc0 = jnp.zeros((QB, MAXNB), jnp.float32)
    prior0 = jnp.zeros((QB, 1), jnp.float32)
    acc, total = lax.fori_loop(0, nch, chunk, (acc0, prior0))
    nb_ref[...] = jnp.minimum(acc, float(N - 1)).astype(jnp.int32)
    valid_ref[...] = (thr < jnp.minimum(total, float(MAXNB)).astype(jnp.int32)
                      ).astype(jnp.float32)


def _radius(pos_dst, pos_src, r, interpret=False):
    M = pos_dst.shape[0]
    N = pos_src.shape[0]
    QB = 128 if M >= 128 else M
    NC = 512 if N >= 512 else N
    nch = N // NC
    src3 = pos_src.T.reshape(3, nch, NC).transpose(1, 0, 2)
    r2 = float(np.float32(np.float64(r) * np.float64(r)))
    nb, valid = _pallas_call(
        functools.partial(_radius_kernel, N=N, NC=NC, r2=r2),
        grid=(M // QB,),
        in_specs=[pl.BlockSpec((QB, 3), lambda i: (i, 0)),
                  pl.BlockSpec((nch, 3, NC), lambda i: (0, 0, 0))],
        out_specs=[pl.BlockSpec((QB, MAXNB), lambda i: (i, 0)),
                   pl.BlockSpec((QB, MAXNB), lambda i: (i, 0))],
        out_shape=(jax.ShapeDtypeStruct((M, MAXNB), jnp.int32),
                   jax.ShapeDtypeStruct((M, MAXNB), jnp.float32)),
        interpret=interpret,
    )(pos_dst, src3)
    return nb, valid


def _sc_gather(table, idx):
    V, D0 = table.shape
    if D0 % 128 != 0:
        table = jnp.pad(table, ((0, 0), (0, 128 - D0 % 128)))
    D = table.shape[1]
    B = idx.shape[0]
    NW = 32
    b_per_w = B // NW
    CH = b_per_w
    while CH > 128 or CH * D * 4 > 262144:
        CH //= 2
    nch = b_per_w // CH
    mesh = plsc.VectorSubcoreMesh(core_axis_name="c", subcore_axis_name="s")

    @functools.partial(
        pl.kernel, mesh=mesh,
        out_type=jax.ShapeDtypeStruct((B, D), jnp.float32),
        scratch_types=[pltpu.VMEM((CH,), jnp.int32),
                       pltpu.VMEM((CH, D), jnp.float32),
                       pltpu.SemaphoreType.DMA],
    )
    def k(table_hbm, idx_hbm, out_hbm, idx_v, rows_v, sem):
        wid = lax.axis_index("s") * 2 + lax.axis_index("c")
        base = wid * b_per_w

        def body(c, carry):
            off = base + c * CH
            pltpu.sync_copy(idx_hbm.at[pl.ds(off, CH)], idx_v)
            pltpu.async_copy(table_hbm.at[idx_v], rows_v, sem).wait()
            pltpu.sync_copy(rows_v, out_hbm.at[pl.ds(off, CH)])
            return carry

        lax.fori_loop(0, nch, body, 0)

    out = k(table, idx)
    return out[:, :D0] if D != D0 else out


def _conv_kernel(gg_ref, posd_ref, w1r_ref, w2_ref, b2_ref, valid_ref, out_ref):
    QB = posd_ref.shape[0]
    dh = gg_ref.shape[1]
    dout = out_ref.shape[1]
    rd = jnp.dot(posd_ref[...], w1r_ref[...], preferred_element_type=jnp.float32)
    g = gg_ref[...].reshape(QB, MAXNB, dh)
    h = jnp.maximum(g - rd[:, None, :], 0.0)
    h2 = jnp.dot(h.reshape(QB * MAXNB, dh), w2_ref[...],
                 preferred_element_type=jnp.float32).reshape(QB, MAXNB, dout)
    v = valid_ref[...]
    h2 = jnp.where(v[:, :, None] > 0.0, h2, _NEG_INF)
    out_ref[...] = jnp.max(h2, axis=1) + b2_ref[...]


def _conv(gg, pos_dst, w1r, w2, b2, valid, interpret=False):
    M = pos_dst.shape[0]
    dh = w2.shape[0]
    dout = w2.shape[1]
    QB = max(8, min(M, 8192 // dh))
    out = _pallas_call(
        _conv_kernel,
        grid=(M // QB,),
        in_specs=[pl.BlockSpec((QB * MAXNB, dh), lambda i: (i, 0)),
                  pl.BlockSpec((QB, 3), lambda i: (i, 0)),
                  pl.BlockSpec((3, dh), lambda i: (0, 0)),
                  pl.BlockSpec((dh, dout), lambda i: (0, 0)),
                  pl.BlockSpec((1, dout), lambda i: (0, 0)),
                  pl.BlockSpec((QB, MAXNB), lambda i: (i, 0))],
        out_specs=pl.BlockSpec((QB, dout), lambda i: (i, 0)),
        out_shape=jax.ShapeDtypeStruct((M, dout), jnp.float32),
        interpret=interpret,
    )(gg, pos_dst, w1r, w2, b2.reshape(1, -1), valid)
    return out


def _knn_kernel(posd_ref, srcT_ref, idx_ref, w_ref, *, S):
    QB = posd_ref.shape[0]
    pd = posd_ref[...]
    qn = jnp.sum(pd * pd, axis=1, keepdims=True)
    st = srcT_ref[...]
    pn = jnp.sum(st * st, axis=0, keepdims=True)
    d2 = qn + pn - 2.0 * jnp.dot(pd, st, preferred_element_type=jnp.float32)
    li = lax.broadcasted_iota(jnp.int32, (QB, S), 1)
    idxs, ws = [], []
    for _ in range(3):
        m = jnp.min(d2, axis=1, keepdims=True)
        ii = jnp.min(jnp.where(d2 == m, li, S), axis=1, keepdims=True)
        idxs.append(ii)
        ws.append(1.0 / jnp.maximum(m, 1e-16))
        d2 = jnp.where(li == ii, jnp.inf, d2)
    idx_ref[...] = jnp.concatenate(idxs + [jnp.zeros((QB, 5), jnp.int32)], axis=1)
    w_ref[...] = jnp.concatenate(ws + [jnp.zeros((QB, 5), jnp.float32)], axis=1)


def _knn(pos_dst, pos_src, interpret=False):
    M = pos_dst.shape[0]
    S = pos_src.shape[0]
    QB = 128 if M >= 128 else M
    idx, w = _pallas_call(
        functools.partial(_knn_kernel, S=S),
        grid=(M // QB,),
        in_specs=[pl.BlockSpec((QB, 3), lambda i: (i, 0)),
                  pl.BlockSpec((3, S), lambda i: (0, 0))],
        out_specs=[pl.BlockSpec((QB, 8), lambda i: (i, 0)),
                   pl.BlockSpec((QB, 8), lambda i: (i, 0))],
        out_shape=(jax.ShapeDtypeStruct((M, 8), jnp.int32),
                   jax.ShapeDtypeStruct((M, 8), jnp.float32)),
        interpret=interpret,
    )(pos_dst, pos_src.T)
    return idx, w


def _wsum_kernel(r0_ref, r1_ref, r2_ref, w_ref, out_ref):
    w = w_ref[...]
    w0 = w[:, 0:1]
    w1 = w[:, 1:2]
    w2 = w[:, 2:3]
    num = r0_ref[...] * w0 + r1_ref[...] * w1 + r2_ref[...] * w2
    out_ref[...] = num / (w0 + w1 + w2)


def _wsum(r0, r1, r2, w, interpret=False):
    M, D = r0.shape
    MB = 128 if M >= 128 else M
    out = _pallas_call(
        _wsum_kernel,
        grid=(M // MB,),
        in_specs=[pl.BlockSpec((MB, D), lambda i: (i, 0)),
                  pl.BlockSpec((MB, D), lambda i: (i, 0)),
                  pl.BlockSpec((MB, D), lambda i: (i, 0)),
                  pl.BlockSpec((MB, 8), lambda i: (i, 0))],
        out_specs=pl.BlockSpec((MB, D), lambda i: (i, 0)),
        out_shape=jax.ShapeDtypeStruct((M, D), jnp.float32),
        interpret=interpret,
    )(r0, r1, r2, w)
    return out


def _linear_kernel(x_ref, w_ref, b_ref, out_ref):
    out_ref[...] = (jnp.dot(x_ref[...], w_ref[...],
                            preferred_element_type=jnp.float32) + b_ref[...])


def _linear(x, w, b, interpret=False):
    M, din = x.shape
    dout = w.shape[1]
    MB = 256 if M >= 256 else M
    out = _pallas_call(
        _linear_kernel,
        grid=(M // MB,),
        in_specs=[pl.BlockSpec((MB, din), lambda i: (i, 0)),
                  pl.BlockSpec((din, dout), lambda i: (0, 0)),
                  pl.BlockSpec((1, dout), lambda i: (0, 0))],
        out_specs=pl.BlockSpec((MB, dout), lambda i: (i, 0)),
        out_shape=jax.ShapeDtypeStruct((M, dout), jnp.float32),
        interpret=interpret,
    )(x, w, b.reshape(1, -1))
    return out


def _mlp_kernel(x_ref, w1_ref, b1_ref, w2_ref, b2_ref, out_ref):
    h = jnp.maximum(jnp.dot(x_ref[...], w1_ref[...],
                            preferred_element_type=jnp.float32) + b1_ref[...], 0.0)
    out_ref[...] = (jnp.dot(h, w2_ref[...],
                            preferred_element_type=jnp.float32) + b2_ref[...])


def _mlp(x, p, interpret=False):
    w1, b1, w2, b2 = p
    M, din = x.shape
    dh = w1.shape[1]
    dout = w2.shape[1]
    MB = 256 if M >= 256 else M
    out = _pallas_call(
        _mlp_kernel,
        grid=(M // MB,),
        in_specs=[pl.BlockSpec((MB, din), lambda i: (i, 0)),
                  pl.BlockSpec((din, dh), lambda i: (0, 0)),
                  pl.BlockSpec((1, dh), lambda i: (0, 0)),
                  pl.BlockSpec((dh, dout), lambda i: (0, 0)),
                  pl.BlockSpec((1, dout), lambda i: (0, 0))],
        out_specs=pl.BlockSpec((MB, dout), lambda i: (i, 0)),
        out_shape=jax.ShapeDtypeStruct((M, dout), jnp.float32),
        interpret=interpret,
    )(x, w1, b1.reshape(1, -1), w2, b2.reshape(1, -1))
    return out


def _sa_layer(p, x, pos, r):
    w1, b1, w2, b2 = p
    N = pos.shape[0]
    n = N // 4
    dx = x.shape[1]
    _, sx, sy, sz = _fps(pos, n)
    pos_dst = jnp.concatenate([sx, sy, sz], axis=1)
    nb, valid = _radius(pos_dst, pos, r)
    g = _linear(jnp.concatenate([x, pos], axis=1), w1, b1)
    gg = _sc_gather(g, nb.reshape(-1))
    out = _conv(gg, pos_dst, w1[dx:], w2, b2, valid)
    return out, pos_dst


def _fp_layer(p, x, pos_src, x_skip, pos_dst):
    idx3, w = _knn(pos_dst, pos_src)
    flat = idx3[:, :3].reshape(-1)
    rows = _sc_gather(x, flat)
    r3 = rows.reshape(-1, 3, x.shape[1])
    xi = _wsum(r3[:, 0, :], r3[:, 1, :], r3[:, 2, :], w)
    return _mlp(jnp.concatenate([xi, x_skip], axis=1), p)


def kernel(x, pos, batch, params):
    x1, pos1 = _sa_layer(params["sa1"], x, pos, 0.1)
    x2, pos2 = _sa_layer(params["sa2"], x1, pos1, 0.2)
    x3, pos3 = _sa_layer(params["sa3"], x2, pos2, 0.4)
    x4, pos4 = _sa_layer(params["sa4"], x3, pos3, 0.8)
    h = _fp_layer(params["fp4"], x4, pos4, x3, pos3)
    h = _fp_layer(params["fp3"], h, pos3, x2, pos2)
    h = _fp_layer(params["fp2"], h, pos2, x1, pos1)
    h = _fp_layer(params["fp1"], h, pos1, x, pos)
    sem = _mlp(h, params["sem"])
    inst = _mlp(h, params["inst"])
    return sem, inst


# speedup vs baseline: 9.0006x; 1.0023x over previous
import functools

import numpy as np
import jax
import jax.numpy as jnp
from jax import lax
from jax.experimental import pallas as pl
from jax.experimental.pallas import tpu as pltpu
from jax.experimental.pallas import tpu_sc as plsc

MAXNB = 64
_NEG_INF = float("-inf")

_pallas_call = pl.pallas_call


def _fps_kernel(px_ref, py_ref, pz_ref, idx_ref, sx_ref, sy_ref, sz_ref, *,
                n, N, R):
    fi = (lax.broadcasted_iota(jnp.int32, (R, 128), 0) * 128
          + lax.broadcasted_iota(jnp.int32, (R, 128), 1))
    lane = lax.broadcasted_iota(jnp.int32, (1, 128), 1)

    def extract(ref, row, col):
        rv = ref[pl.ds(row, 1), :]
        return jnp.sum(jnp.where(lane == col, rv, 0.0))

    px = px_ref[...]
    py = py_ref[...]
    pz = pz_ref[...]

    lx0 = extract(px_ref, 0, 0)
    ly0 = extract(py_ref, 0, 0)
    lz0 = extract(pz_ref, 0, 0)
    idx_ref[pl.ds(0, 1), :] = jnp.zeros((1, 1), jnp.int32)
    sx_ref[pl.ds(0, 1), :] = jnp.reshape(lx0, (1, 1))
    sy_ref[pl.ds(0, 1), :] = jnp.reshape(ly0, (1, 1))
    sz_ref[pl.ds(0, 1), :] = jnp.reshape(lz0, (1, 1))

    def body(i, carry):
        dmin, lx, ly, lz = carry
        dx = px - lx
        dy = py - ly
        dz = pz - lz
        d = dx * dx + dy * dy + dz * dz
        dmin = jnp.minimum(dmin, d)
        m = jnp.max(dmin)
        bid = jnp.min(jnp.where(dmin == m, fi, N))
        idx_ref[pl.ds(i, 1), :] = jnp.reshape(bid, (1, 1))
        row = bid // 128
        col = bid % 128
        nlx = extract(px_ref, row, col)
        nly = extract(py_ref, row, col)
        nlz = extract(pz_ref, row, col)
        sx_ref[pl.ds(i, 1), :] = jnp.reshape(nlx, (1, 1))
        sy_ref[pl.ds(i, 1), :] = jnp.reshape(nly, (1, 1))
        sz_ref[pl.ds(i, 1), :] = jnp.reshape(nlz, (1, 1))
        return dmin, nlx, nly, nlz

    dmin0 = jnp.full((R, 128), jnp.inf, jnp.float32)
    lax.fori_loop(1, n, body, (dmin0, lx0, ly0, lz0))


def _fps(pos, n, interpret=False):
    N = pos.shape[0]
    R = N // 128
    px = pos[:, 0].reshape(R, 128)
    py = pos[:, 1].reshape(R, 128)
    pz = pos[:, 2].reshape(R, 128)
    outs = _pallas_call(
        functools.partial(_fps_kernel, n=n, N=N, R=R),
        out_shape=(jax.ShapeDtypeStruct((n, 1), jnp.int32),
                   jax.ShapeDtypeStruct((n, 1), jnp.float32),
                   jax.ShapeDtypeStruct((n, 1), jnp.float32),
                   jax.ShapeDtypeStruct((n, 1), jnp.float32)),
        interpret=interpret,
    )(px, py, pz)
    return outs


def _radius_kernel(posd_ref, src3_ref, nb_ref, valid_ref, *, N, NC, r2):
    QB = posd_ref.shape[0]
    pd = posd_ref[...]
    qn = jnp.sum(pd * pd, axis=1, keepdims=True)
    nch = N // NC
    U = (lax.broadcasted_iota(jnp.int32, (NC, NC), 0)
         <= lax.broadcasted_iota(jnp.int32, (NC, NC), 1)).astype(jnp.float32)
    thr = lax.broadcasted_iota(jnp.int32, (1, MAXNB), 1)

    def chunk(c, carry):
        acc, prior = carry
        st = src3_ref[pl.ds(c, 1), :, :].reshape(3, NC)
        pn = jnp.sum(st * st, axis=0, keepdims=True)
        d2 = qn + pn - 2.0 * jnp.dot(pd, st, preferred_element_type=jnp.float32)
        win = (d2 <= r2).astype(jnp.float32)
        cum = prior + jnp.dot(win, U, precision=lax.Precision.HIGHEST,
                              preferred_element_type=jnp.float32)
        cols = [jnp.sum((cum <= j).astype(jnp.float32), axis=1, keepdims=True)
                for j in range(MAXNB)]
        acc = acc + jnp.concatenate(cols, axis=1)
        prior = cum[:, NC - 1:NC]
        return acc, prior

    acc0 = jnp.zeros((QB, MAXNB), jnp.float32)
    prior0 = jnp.zeros((QB, 1), jnp.float32)
    acc, total = lax.fori_loop(0, nch, chunk, (acc0, prior0))
    nb_ref[...] = jnp.minimum(acc, float(N - 1)).astype(jnp.int32)
    valid_ref[...] = (thr < jnp.minimum(total, float(MAXNB)).astype(jnp.int32)
                      ).astype(jnp.float32)


def _radius(pos_dst, pos_src, r, interpret=False):
    M = pos_dst.shape[0]
    N = pos_src.shape[0]
    QB = 128 if M >= 128 else M
    NC = 512 if N >= 512 else N
    nch = N // NC
    src3 = pos_src.T.reshape(3, nch, NC).transpose(1, 0, 2)
    r2 = float(np.float32(np.float64(r) * np.float64(r)))
    nb, valid = _pallas_call(
        functools.partial(_radius_kernel, N=N, NC=NC, r2=r2),
        grid=(M // QB,),
        in_specs=[pl.BlockSpec((QB, 3), lambda i: (i, 0)),
                  pl.BlockSpec((nch, 3, NC), lambda i: (0, 0, 0))],
        out_specs=[pl.BlockSpec((QB, MAXNB), lambda i: (i, 0)),
                   pl.BlockSpec((QB, MAXNB), lambda i: (i, 0))],
        out_shape=(jax.ShapeDtypeStruct((M, MAXNB), jnp.int32),
                   jax.ShapeDtypeStruct((M, MAXNB), jnp.float32)),
        interpret=interpret,
    )(pos_dst, src3)
    return nb, valid


def _sc_gather(table, idx):
    V, D0 = table.shape
    if D0 % 128 != 0:
        table = jnp.pad(table, ((0, 0), (0, 128 - D0 % 128)))
    D = table.shape[1]
    B = idx.shape[0]
    NW = 32
    b_per_w = B // NW
    mesh = plsc.VectorSubcoreMesh(core_axis_name="c", subcore_axis_name="s")

    if b_per_w % 128 == 0:
        rpw = b_per_w // 128
        K = min(4, max(1, 262144 // (128 * D * 4)), rpw)
        while rpw % K:
            K -= 1
        ng = rpw // K

        @functools.partial(
            pl.kernel, mesh=mesh,
            out_type=jax.ShapeDtypeStruct((B, D), jnp.float32),
            scratch_types=[pltpu.VMEM((K * 128,), jnp.int32),
                           pltpu.VMEM((K * 128, D), jnp.float32),
                           pltpu.SemaphoreType.DMA],
        )
        def k2(table_hbm, idx_hbm, out_hbm, idx_v, rows_v, sem):
            wid = lax.axis_index("s") * 2 + lax.axis_index("c")
            base = wid * b_per_w

            def body(g, carry):
                off = base + g * (K * 128)
                pltpu.sync_copy(idx_hbm.at[pl.ds(off, K * 128)], idx_v)
                for j in range(K):
                    pltpu.async_copy(table_hbm.at[idx_v.at[pl.ds(j * 128, 128)]],
                                     rows_v.at[pl.ds(j * 128, 128)], sem)
                for j in range(K):
                    pltpu.make_async_copy(table_hbm.at[idx_v.at[pl.ds(j * 128, 128)]],
                                          rows_v.at[pl.ds(j * 128, 128)],
                                          sem).wait()
                pltpu.sync_copy(rows_v, out_hbm.at[pl.ds(off, K * 128)])
                return carry

            lax.fori_loop(0, ng, body, 0)

        out = k2(table, idx)
        return out[:, :D0] if D != D0 else out

    CH = b_per_w
    while CH > 128 or CH * D * 4 > 262144:
        CH //= 2
    nch = b_per_w // CH

    @functools.partial(
        pl.kernel, mesh=mesh,
        out_type=jax.ShapeDtypeStruct((B, D), jnp.float32),
        scratch_types=[pltpu.VMEM((CH,), jnp.int32),
                       pltpu.VMEM((CH, D), jnp.float32),
                       pltpu.SemaphoreType.DMA],
    )
    def k(table_hbm, idx_hbm, out_hbm, idx_v, rows_v, sem):
        wid = lax.axis_index("s") * 2 + lax.axis_index("c")
        base = wid * b_per_w

        def body(c, carry):
            off = base + c * CH
            pltpu.sync_copy(idx_hbm.at[pl.ds(off, CH)], idx_v)
            pltpu.async_copy(table_hbm.at[idx_v], rows_v, sem).wait()
            pltpu.sync_copy(rows_v, out_hbm.at[pl.ds(off, CH)])
            return carry

        lax.fori_loop(0, nch, body, 0)

    out = k(table, idx)
    return out[:, :D0] if D != D0 else out


def _conv_kernel(gg_ref, posd_ref, w1r_ref, w2_ref, b2_ref, valid_ref, out_ref):
    QB = posd_ref.shape[0]
    dh = gg_ref.shape[1]
    dout = out_ref.shape[1]
    rd = jnp.dot(posd_ref[...], w1r_ref[...], preferred_element_type=jnp.float32)
    g = gg_ref[...].reshape(QB, MAXNB, dh)
    h = jnp.maximum(g - rd[:, None, :], 0.0)
    h2 = jnp.dot(h.reshape(QB * MAXNB, dh), w2_ref[...],
                 preferred_element_type=jnp.float32).reshape(QB, MAXNB, dout)
    v = valid_ref[...]
    h2 = jnp.where(v[:, :, None] > 0.0, h2, _NEG_INF)
    out_ref[...] = jnp.max(h2, axis=1) + b2_ref[...]


def _conv(gg, pos_dst, w1r, w2, b2, valid, interpret=False):
    M = pos_dst.shape[0]
    dh = w2.shape[0]
    dout = w2.shape[1]
    QB = max(8, min(M, 8192 // dh))
    out = _pallas_call(
        _conv_kernel,
        grid=(M // QB,),
        in_specs=[pl.BlockSpec((QB * MAXNB, dh), lambda i: (i, 0)),
                  pl.BlockSpec((QB, 3), lambda i: (i, 0)),
                  pl.BlockSpec((3, dh), lambda i: (0, 0)),
                  pl.BlockSpec((dh, dout), lambda i: (0, 0)),
                  pl.BlockSpec((1, dout), lambda i: (0, 0)),
                  pl.BlockSpec((QB, MAXNB), lambda i: (i, 0))],
        out_specs=pl.BlockSpec((QB, dout), lambda i: (i, 0)),
        out_shape=jax.ShapeDtypeStruct((M, dout), jnp.float32),
        interpret=interpret,
    )(gg, pos_dst, w1r, w2, b2.reshape(1, -1), valid)
    return out


def _knn_kernel(posd_ref, srcT_ref, idx_ref, w_ref, *, S):
    QB = posd_ref.shape[0]
    pd = posd_ref[...]
    qn = jnp.sum(pd * pd, axis=1, keepdims=True)
    st = srcT_ref[...]
    pn = jnp.sum(st * st, axis=0, keepdims=True)
    d2 = qn + pn - 2.0 * jnp.dot(pd, st, preferred_element_type=jnp.float32)
    li = lax.broadcasted_iota(jnp.int32, (QB, S), 1)
    idxs, ws = [], []
    for _ in range(3):
        m = jnp.min(d2, axis=1, keepdims=True)
        ii = jnp.min(jnp.where(d2 == m, li, S), axis=1, keepdims=True)
        idxs.append(ii)
        ws.append(1.0 / jnp.maximum(m, 1e-16))
        d2 = jnp.where(li == ii, jnp.inf, d2)
    idx_ref[...] = jnp.concatenate(idxs + [jnp.zeros((QB, 5), jnp.int32)], axis=1)
    w_ref[...] = jnp.concatenate(ws + [jnp.zeros((QB, 5), jnp.float32)], axis=1)


def _knn(pos_dst, pos_src, interpret=False):
    M = pos_dst.shape[0]
    S = pos_src.shape[0]
    QB = 128 if M >= 128 else M
    idx, w = _pallas_call(
        functools.partial(_knn_kernel, S=S),
        grid=(M // QB,),
        in_specs=[pl.BlockSpec((QB, 3), lambda i: (i, 0)),
                  pl.BlockSpec((3, S), lambda i: (0, 0))],
        out_specs=[pl.BlockSpec((QB, 8), lambda i: (i, 0)),
                   pl.BlockSpec((QB, 8), lambda i: (i, 0))],
        out_shape=(jax.ShapeDtypeStruct((M, 8), jnp.int32),
                   jax.ShapeDtypeStruct((M, 8), jnp.float32)),
        interpret=interpret,
    )(pos_dst, pos_src.T)
    return idx, w


def _wsum_kernel(r0_ref, r1_ref, r2_ref, w_ref, out_ref):
    w = w_ref[...]
    w0 = w[:, 0:1]
    w1 = w[:, 1:2]
    w2 = w[:, 2:3]
    num = r0_ref[...] * w0 + r1_ref[...] * w1 + r2_ref[...] * w2
    out_ref[...] = num / (w0 + w1 + w2)


def _wsum(r0, r1, r2, w, interpret=False):
    M, D = r0.shape
    MB = 128 if M >= 128 else M
    out = _pallas_call(
        _wsum_kernel,
        grid=(M // MB,),
        in_specs=[pl.BlockSpec((MB, D), lambda i: (i, 0)),
                  pl.BlockSpec((MB, D), lambda i: (i, 0)),
                  pl.BlockSpec((MB, D), lambda i: (i, 0)),
                  pl.BlockSpec((MB, 8), lambda i: (i, 0))],
        out_specs=pl.BlockSpec((MB, D), lambda i: (i, 0)),
        out_shape=jax.ShapeDtypeStruct((M, D), jnp.float32),
        interpret=interpret,
    )(r0, r1, r2, w)
    return out


def _linear_kernel(x_ref, w_ref, b_ref, out_ref):
    out_ref[...] = (jnp.dot(x_ref[...], w_ref[...],
                            preferred_element_type=jnp.float32) + b_ref[...])


def _linear(x, w, b, interpret=False):
    M, din = x.shape
    dout = w.shape[1]
    MB = 256 if M >= 256 else M
    out = _pallas_call(
        _linear_kernel,
        grid=(M // MB,),
        in_specs=[pl.BlockSpec((MB, din), lambda i: (i, 0)),
                  pl.BlockSpec((din, dout), lambda i: (0, 0)),
                  pl.BlockSpec((1, dout), lambda i: (0, 0))],
        out_specs=pl.BlockSpec((MB, dout), lambda i: (i, 0)),
        out_shape=jax.ShapeDtypeStruct((M, dout), jnp.float32),
        interpret=interpret,
    )(x, w, b.reshape(1, -1))
    return out


def _mlp_kernel(x_ref, w1_ref, b1_ref, w2_ref, b2_ref, out_ref):
    h = jnp.maximum(jnp.dot(x_ref[...], w1_ref[...],
                            preferred_element_type=jnp.float32) + b1_ref[...], 0.0)
    out_ref[...] = (jnp.dot(h, w2_ref[...],
                            preferred_element_type=jnp.float32) + b2_ref[...])


def _mlp(x, p, interpret=False):
    w1, b1, w2, b2 = p
    M, din = x.shape
    dh = w1.shape[1]
    dout = w2.shape[1]
    MB = 256 if M >= 256 else M
    out = _pallas_call(
        _mlp_kernel,
        grid=(M // MB,),
        in_specs=[pl.BlockSpec((MB, din), lambda i: (i, 0)),
                  pl.BlockSpec((din, dh), lambda i: (0, 0)),
                  pl.BlockSpec((1, dh), lambda i: (0, 0)),
                  pl.BlockSpec((dh, dout), lambda i: (0, 0)),
                  pl.BlockSpec((1, dout), lambda i: (0, 0))],
        out_specs=pl.BlockSpec((MB, dout), lambda i: (i, 0)),
        out_shape=jax.ShapeDtypeStruct((M, dout), jnp.float32),
        interpret=interpret,
    )(x, w1, b1.reshape(1, -1), w2, b2.reshape(1, -1))
    return out


def _sa_layer(p, x, pos, r):
    w1, b1, w2, b2 = p
    N = pos.shape[0]
    n = N // 4
    dx = x.shape[1]
    _, sx, sy, sz = _fps(pos, n)
    pos_dst = jnp.concatenate([sx, sy, sz], axis=1)
    nb, valid = _radius(pos_dst, pos, r)
    g = _linear(jnp.concatenate([x, pos], axis=1), w1, b1)
    gg = _sc_gather(g, nb.reshape(-1))
    out = _conv(gg, pos_dst, w1[dx:], w2, b2, valid)
    return out, pos_dst


def _fp_layer(p, x, pos_src, x_skip, pos_dst):
    idx3, w = _knn(pos_dst, pos_src)
    flat = idx3[:, :3].reshape(-1)
    rows = _sc_gather(x, flat)
    r3 = rows.reshape(-1, 3, x.shape[1])
    xi = _wsum(r3[:, 0, :], r3[:, 1, :], r3[:, 2, :], w)
    return _mlp(jnp.concatenate([xi, x_skip], axis=1), p)


def kernel(x, pos, batch, params):
    x1, pos1 = _sa_layer(params["sa1"], x, pos, 0.1)
    x2, pos2 = _sa_layer(params["sa2"], x1, pos1, 0.2)
    x3, pos3 = _sa_layer(params["sa3"], x2, pos2, 0.4)
    x4, pos4 = _sa_layer(params["sa4"], x3, pos3, 0.8)
    h = _fp_layer(params["fp4"], x4, pos4, x3, pos3)
    h = _fp_layer(params["fp3"], h, pos3, x2, pos2)
    h = _fp_layer(params["fp2"], h, pos2, x1, pos1)
    h = _fp_layer(params["fp1"], h, pos1, x, pos)
    sem = _mlp(h, params["sem"])
    inst = _mlp(h, params["inst"])
    return sem, inst
